# Initial kernel scaffold; baseline (speedup 1.0000x reference)
import jax
import jax.numpy as jnp
from jax.experimental import pallas as pl


def kernel(genes, scales, table):
    raise NotImplementedError("write your pallas kernel here")


# speedup vs baseline: 3.7228x; 3.7228x over previous
import functools

import jax
import jax.numpy as jnp
from jax import lax
from jax.experimental import pallas as pl
from jax.experimental.pallas import tpu as pltpu
from jax.experimental.pallas import tpu_sc as plsc

EPS = 1e-12
NUM_CORES = 2
NUM_SUBCORES = 16
NUM_WORKERS = NUM_CORES * NUM_SUBCORES
LANES = 16
IDX_MINOR = 128


def _tc_normalize_table(table):
    v, d = table.shape
    block = 2000
    assert v % block == 0

    def body(t_ref, o_ref):
        x = t_ref[...]
        ss = jnp.sum(x * x, axis=1, keepdims=True)
        y = (x + EPS) / (jnp.sqrt(ss) + EPS)
        o_ref[...] = jnp.concatenate([y, jnp.zeros_like(y)], axis=1)

    return pl.pallas_call(
        body,
        grid=(v // block,),
        in_specs=[pl.BlockSpec((block, d), lambda i: (i, 0))],
        out_specs=pl.BlockSpec((block, 2 * d), lambda i: (i, 0)),
        out_shape=jax.ShapeDtypeStruct((v, 2 * d), jnp.float32),
    )(table)


@functools.partial(jax.jit, static_argnames=("n_rows", "d", "chunk"))
def _sc_gather_scale(table_p, genes_flat, scales_flat, n_rows, d, chunk):
    rows_per_worker = n_rows // NUM_WORKERS
    n_chunks = rows_per_worker // chunk
    idx_rows = chunk // IDX_MINOR
    dp = table_p.shape[1]

    mesh = plsc.VectorSubcoreMesh(core_axis_name="c", subcore_axis_name="s")

    @functools.partial(
        pl.kernel,
        mesh=mesh,
        out_type=jax.ShapeDtypeStruct((n_rows, d), jnp.float32),
        scratch_types=[
            pltpu.VMEM((chunk,), jnp.int32),
            pltpu.VMEM((chunk, dp), jnp.float32),
            pltpu.VMEM((chunk, d), jnp.float32),
            pltpu.VMEM((chunk,), jnp.float32),
            pltpu.SemaphoreType.DMA,
        ],
    )
    def k(table_hbm, genes_hbm, scales_hbm, out_hbm,
          idx_v, rows_v, out_v, scl_v, sem):
        wid = lax.axis_index("s") * NUM_CORES + lax.axis_index("c")
        base = wid * rows_per_worker

        def chunk_body(c, carry):
            off = pl.multiple_of(base + c * chunk, chunk)
            pltpu.sync_copy(genes_hbm.at[pl.ds(off, chunk)], idx_v)
            pltpu.sync_copy(scales_hbm.at[pl.ds(off, chunk)], scl_v)
            copies = [
                pltpu.async_copy(
                    table_hbm.at[idx_v.at[pl.ds(j * IDX_MINOR, IDX_MINOR)]],
                    rows_v.at[pl.ds(j * IDX_MINOR, IDX_MINOR)],
                    sem,
                )
                for j in range(idx_rows)
            ]
            for cp in copies:
                cp.wait()

            def group_body(g, carry2):
                row0 = pl.multiple_of(g * LANES, LANES)
                scl16 = scl_v[pl.ds(row0, LANES)]
                for r in range(LANES):
                    s = scl16[r]
                    for q in range(d // LANES):
                        sl = pl.ds(q * LANES, LANES)
                        out_v[row0 + r, sl] = rows_v[row0 + r, sl] * s
                return carry2

            lax.fori_loop(0, chunk // LANES, group_body, 0)
            pltpu.sync_copy(out_v, out_hbm.at[pl.ds(off, chunk)])
            return carry

        lax.fori_loop(0, n_chunks, chunk_body, 0)

    return k(table_p, genes_flat, scales_flat)


def kernel(genes, scales, table):
    b, l = genes.shape
    v, d = table.shape
    n_rows = b * l
    table_p = _tc_normalize_table(table.astype(jnp.float32))
    genes_flat = genes.reshape(-1).astype(jnp.int32)
    scales_flat = scales.reshape(-1).astype(jnp.float32)
    out = _sc_gather_scale(table_p, genes_flat, scales_flat, n_rows, d, 256)
    return out.reshape(b, l, d)
